# Initial kernel scaffold; baseline (speedup 1.0000x reference)
import jax
import jax.numpy as jnp
from jax.experimental import pallas as pl


def kernel(adj, embeds, qTrans, kTrans, vTrans, filt):
    raise NotImplementedError("write your pallas kernel here")


# speedup vs baseline: 1.0533x; 1.0533x over previous
import functools

import jax
import jax.numpy as jnp
from jax import lax
from jax.experimental import pallas as pl

N_NODES = 10000
N_EDGES = 160000
LATDIM = 256
HEAD = 4
DH = LATDIM // HEAD

_NODE_BLK = 1024
_EDGE_BLK = 2048


def _qkv_body(e_ref, wq_ref, wk_ref, wv_ref, q_ref, k_ref, v_ref):
    e = e_ref[...]
    q_ref[...] = jnp.dot(e, wq_ref[...], preferred_element_type=jnp.float32)
    k_ref[...] = jnp.dot(e, wk_ref[...], preferred_element_type=jnp.float32)
    v_ref[...] = jnp.dot(e, wv_ref[...], preferred_element_type=jnp.float32)


def _qkv(embeds, qT, kT, vT):
    n = embeds.shape[0]
    grid = (pl.cdiv(n, _NODE_BLK),)
    spec_e = pl.BlockSpec((_NODE_BLK, LATDIM), lambda i: (i, 0))
    spec_w = pl.BlockSpec((LATDIM, LATDIM), lambda i: (0, 0))
    out = jax.ShapeDtypeStruct((n, LATDIM), jnp.float32)
    return pl.pallas_call(
        _qkv_body,
        grid=grid,
        in_specs=[spec_e, spec_w, spec_w, spec_w],
        out_specs=[spec_e, spec_e, spec_e],
        out_shape=[out, out, out],
    )(embeds, qT, kT, vT)


def _score_body(q_ref, k_ref, f_ref, o_ref):
    prod = q_ref[...] * k_ref[...]
    cols = []
    for h in range(HEAD):
        s = jnp.sum(prod[:, h * DH:(h + 1) * DH], axis=1, keepdims=True)
        cols.append(s)
    att = jnp.concatenate(cols, axis=1)
    att = jnp.clip(att, -10.0, 10.0) + f_ref[:, :HEAD]
    o_ref[...] = jnp.exp(att)


def _scores(qr, kc, fc):
    e = qr.shape[0]
    grid = (pl.cdiv(e, _EDGE_BLK),)
    spec_qk = pl.BlockSpec((_EDGE_BLK, LATDIM), lambda i: (i, 0))
    spec_f = pl.BlockSpec((_EDGE_BLK, HEAD), lambda i: (i, 0))
    return pl.pallas_call(
        _score_body,
        grid=grid,
        in_specs=[spec_qk, spec_qk, spec_f],
        out_specs=spec_f,
        out_shape=jax.ShapeDtypeStruct((e, HEAD), jnp.float32),
    )(qr, kc, fc)


def _weight_body(a_ref, n_ref, v_ref, o_ref):
    att = a_ref[...] / (n_ref[...] + 1e-8)
    v = v_ref[...]
    cols = []
    for h in range(HEAD):
        cols.append(att[:, h:h + 1] * v[:, h * DH:(h + 1) * DH])
    o_ref[...] = jnp.concatenate(cols, axis=1)


def _weighted_v(expatt, normr, vc):
    e = vc.shape[0]
    grid = (pl.cdiv(e, _EDGE_BLK),)
    spec_a = pl.BlockSpec((_EDGE_BLK, HEAD), lambda i: (i, 0))
    spec_v = pl.BlockSpec((_EDGE_BLK, LATDIM), lambda i: (i, 0))
    return pl.pallas_call(
        _weight_body,
        grid=grid,
        in_specs=[spec_a, spec_a, spec_v],
        out_specs=spec_v,
        out_shape=jax.ShapeDtypeStruct((e, LATDIM), jnp.float32),
    )(expatt, normr, vc)


def kernel(adj, embeds, qTrans, kTrans, vTrans, filt):
    rows = adj[0, :]
    cols = adj[1, :]
    n = embeds.shape[0]

    q, k, v = _qkv(embeds, qTrans, kTrans, vTrans)

    qr = q[rows]
    kc = k[cols]
    vc = v[cols]
    fc = filt[cols]

    expatt = _scores(qr, kc, fc)

    attnorm = jnp.zeros((n, HEAD), jnp.float32).at[rows].add(expatt)
    normr = attnorm[rows]

    res = _weighted_v(expatt, normr, vc)
    out = jnp.zeros((n, LATDIM), jnp.float32).at[rows].add(res)
    return out
